# baseline (speedup 1.0000x reference)
import functools

import jax
import jax.numpy as jnp
from jax import lax
from jax.experimental import pallas as pl
from jax.experimental.pallas import tpu as pltpu
from jax.experimental.pallas import tpu_sc as plsc

_LANES = 16
_NUM_DET = 100
_SEL0 = 100
_BASE = 96
_NWORK = 7
_NC = 2
_ROWS = 84
_W = _NWORK * _LANES


def _splat(cm_v, k):
    return cm_v[pl.ds(k * _LANES, _LANES)]


@functools.partial(
    pl.kernel,
    out_type=jax.ShapeDtypeStruct((_NWORK * 128,), jnp.float32),
    mesh=plsc.VectorSubcoreMesh(core_axis_name="c", subcore_axis_name="s"),
    scratch_types=[
        pltpu.VMEM((_ROWS * 128,), jnp.float32),
        pltpu.VMEM((16 * _LANES,), jnp.float32),
        pltpu.VMEM((128,), jnp.float32),
    ],
    compiler_params=pltpu.CompilerParams(needs_layout_passes=False),
)
def _sc_detect(x_hbm, cm_hbm, out_hbm, xv, cmv, outv):
    wid = lax.axis_index("s") * _NC + lax.axis_index("c")

    @pl.when(wid < _NWORK)
    def _():
        pltpu.sync_copy(x_hbm, xv)
        pltpu.sync_copy(cm_hbm, cmv)
        col = wid * _LANES

        best = xv[pl.ds(4 * 128 + col, _LANES)]
        best_id = jnp.zeros((_LANES,), jnp.float32)
        for c in range(1, _ROWS - 4):
            s = xv[pl.ds((4 + c) * 128 + col, _LANES)]
            pr = s > best
            best = jnp.where(pr, s, best)
            best_id = jnp.where(pr, jnp.full((_LANES,), float(c)), best_id)

        b = tuple(xv[pl.ds(i * 128 + col, _LANES)] for i in range(4))
        outv[pl.ds(0, _LANES)] = jnp.zeros((_LANES,), jnp.float32)
        for j in range(4):
            acc = b[0] * _splat(cmv, j)
            for i in range(1, 4):
                acc = acc + b[i] * _splat(cmv, i * 4 + j)
            outv[pl.ds((1 + j) * _LANES, _LANES)] = acc
        outv[pl.ds(5 * _LANES, _LANES)] = best_id
        outv[pl.ds(6 * _LANES, _LANES)] = best
        outv[pl.ds(7 * _LANES, _LANES)] = jnp.zeros((_LANES,), jnp.float32)

        pltpu.sync_copy(outv, out_hbm.at[pl.ds(wid * 128, 128)])


def kernel(x, convert_matrix):
    x2 = x.reshape(x.shape[1], x.shape[2])
    slab = x2[:, _BASE:_BASE + 128].reshape(-1)
    cmf = jnp.tile(convert_matrix.reshape(16, 1), (1, _LANES)).reshape(-1)
    staged = _sc_detect(slab, cmf)
    t = staged.reshape(_NWORK, 8, _LANES)[:, :7, :]
    t = jnp.transpose(t, (0, 2, 1)).reshape(_W, 7)
    off = _SEL0 - _BASE
    return t[off:off + _NUM_DET]
